# baseline (speedup 1.0000x reference)
import jax
import jax.numpy as jnp
import numpy as np
from jax.experimental import pallas as pl
from jax.experimental.pallas import tpu as pltpu

_BR = 400
_BC = 1280
_CAP = 8960
_TW = 1040


def _pass_a_kernel(adj_ref, x_ref, w1_ref, b1_ref, w2_ref,
                   g_ref, part_ref, a_scr, g_scr):
    i = pl.program_id(0)

    @pl.when(i == 0)
    def _():
        a_scr[...] = jnp.dot(x_ref[...], w1_ref[...],
                             preferred_element_type=jnp.float32)

    adj = adj_ref[...]
    h = jnp.dot(adj, a_scr[...], preferred_element_type=jnp.float32)
    h = jnp.maximum(h + b1_ref[...], 0.0)
    g_i = jnp.dot(h, w2_ref[...], preferred_element_type=jnp.float32)
    g_scr[pl.ds(i * _BR, _BR), :] = g_i
    g_ref[...] = g_i

    f_i = (((i + 1) * _BR) // _BC) * _BC
    rows = jax.lax.broadcasted_iota(jnp.int32, g_scr.shape, 0)
    g_z = jnp.where(rows < f_i, g_scr[...], 0.0)
    part_ref[...] = jnp.dot(adj, g_z, preferred_element_type=jnp.float32)


def _pass_b1_kernel(itb, ctb, ftb, adj_hbm, g_ref, part_ref, o_ref,
                    buf, sems):
    t = pl.program_id(0)
    nsteps = pl.num_programs(0)
    slot = jax.lax.rem(t, 2)
    nxt = jax.lax.rem(t + 1, 2)

    def _copy(tt, s):
        r0 = pl.multiple_of(itb[tt] * _BR, 8)
        c0 = pl.multiple_of(ctb[tt], 128)
        return pltpu.make_async_copy(
            adj_hbm.at[pl.ds(r0, _BR), pl.ds(c0, _BC)],
            buf.at[s], sems.at[s])

    @pl.when(t == 0)
    def _():
        _copy(0, 0).start()

    @pl.when(t + 1 < nsteps)
    def _():
        _copy(t + 1, nxt).start()

    _copy(t, slot).wait()

    contrib = jnp.dot(buf[slot],
                      g_ref[pl.ds(pl.multiple_of(ctb[t], 128), _BC), :],
                      preferred_element_type=jnp.float32)
    base = jnp.where(ftb[t] == 1, part_ref[...], o_ref[...])
    o_ref[...] = base + contrib


def _pass_b2_kernel(adj_hbm, g_ref, acc_ref, part_ref, b2_ref, o_ref,
                    buf, sems):
    t = pl.program_id(0)
    nsteps = pl.num_programs(0)
    slot = jax.lax.rem(t, 2)
    nxt = jax.lax.rem(t + 1, 2)

    def _copy(tt, s):
        r0 = pl.multiple_of(tt * _BR, 8)
        return pltpu.make_async_copy(
            adj_hbm.at[pl.ds(r0, _BR), pl.ds(_CAP, _TW)],
            buf.at[s], sems.at[s])

    @pl.when(t == 0)
    def _():
        _copy(0, 0).start()

    @pl.when(t + 1 < nsteps)
    def _():
        _copy(t + 1, nxt).start()

    _copy(t, slot).wait()

    contrib = jnp.dot(buf[slot], g_ref[pl.ds(_CAP, _TW), :],
                      preferred_element_type=jnp.float32)
    no_bulk = (((t + 1) * _BR) // _BC) * _BC == _CAP
    base = jnp.where(no_bulk, part_ref[...], acc_ref[...])
    z = jnp.maximum(base + contrib + b2_ref[...], 0.0)
    m = jnp.max(z, axis=1, keepdims=True)
    s = z - m
    lse = jnp.log(jnp.sum(jnp.exp(s), axis=1, keepdims=True))
    o_ref[...] = s - lse


def _bulk_tables(n):
    nb = n // _BR
    it, ct, ft = [], [], []
    for i in range(nb):
        f_i = (((i + 1) * _BR) // _BC) * _BC
        cols = list(range(f_i, _CAP, _BC))
        for k, c in enumerate(cols):
            it.append(i)
            ct.append(c)
            ft.append(1 if k == 0 else 0)
    return (np.asarray(it, np.int32), np.asarray(ct, np.int32),
            np.asarray(ft, np.int32))


def kernel(x, adj, W1, b1, W2, b2):
    n, d_in = x.shape
    hid = W1.shape[1]
    classes = W2.shape[1]
    b1r = b1.reshape(1, hid)
    b2r = b2.reshape(1, classes)

    nb = n // _BR
    g, part = pl.pallas_call(
        _pass_a_kernel,
        grid=(nb,),
        in_specs=[
            pl.BlockSpec((_BR, n), lambda i: (i, 0)),
            pl.BlockSpec((n, d_in), lambda i: (0, 0)),
            pl.BlockSpec((d_in, hid), lambda i: (0, 0)),
            pl.BlockSpec((1, hid), lambda i: (0, 0)),
            pl.BlockSpec((hid, classes), lambda i: (0, 0)),
        ],
        out_specs=[
            pl.BlockSpec((_BR, classes), lambda i: (i, 0)),
            pl.BlockSpec((_BR, classes), lambda i: (i, 0)),
        ],
        out_shape=[
            jax.ShapeDtypeStruct((n, classes), jnp.float32),
            jax.ShapeDtypeStruct((n, classes), jnp.float32),
        ],
        scratch_shapes=[
            pltpu.VMEM((n, hid), jnp.float32),
            pltpu.VMEM((n, classes), jnp.float32),
        ],
    )(adj, x, W1, b1r, W2)

    it, ct, ft = _bulk_tables(n)
    acc = pl.pallas_call(
        _pass_b1_kernel,
        grid_spec=pltpu.PrefetchScalarGridSpec(
            num_scalar_prefetch=3,
            grid=(it.shape[0],),
            in_specs=[
                pl.BlockSpec(memory_space=pl.ANY),
                pl.BlockSpec((n, classes), lambda t, it, ct, ft: (0, 0)),
                pl.BlockSpec((_BR, classes),
                             lambda t, it, ct, ft: (it[t], 0)),
            ],
            out_specs=pl.BlockSpec((_BR, classes),
                                   lambda t, it, ct, ft: (it[t], 0)),
            scratch_shapes=[
                pltpu.VMEM((2, _BR, _BC), jnp.float32),
                pltpu.SemaphoreType.DMA((2,)),
            ],
        ),
        out_shape=jax.ShapeDtypeStruct((n, classes), jnp.float32),
    )(jnp.asarray(it), jnp.asarray(ct), jnp.asarray(ft), adj, g, part)

    out = pl.pallas_call(
        _pass_b2_kernel,
        grid=(nb,),
        in_specs=[
            pl.BlockSpec(memory_space=pl.ANY),
            pl.BlockSpec((n, classes), lambda t: (0, 0)),
            pl.BlockSpec((_BR, classes), lambda t: (t, 0)),
            pl.BlockSpec((_BR, classes), lambda t: (t, 0)),
            pl.BlockSpec((1, classes), lambda t: (0, 0)),
        ],
        out_specs=pl.BlockSpec((_BR, classes), lambda t: (t, 0)),
        out_shape=jax.ShapeDtypeStruct((n, classes), jnp.float32),
        scratch_shapes=[
            pltpu.VMEM((2, _BR, _TW), jnp.float32),
            pltpu.SemaphoreType.DMA((2,)),
        ],
    )(adj, g, acc, part, b2r)
    return out


# speedup vs baseline: 1.5612x; 1.5612x over previous
import jax
import jax.numpy as jnp
import numpy as np
from jax.experimental import pallas as pl
from jax.experimental.pallas import tpu as pltpu

_BR = 400
_BC = 1280
_CAP = 8960
_TW = 1040
_NSLOT = 4


def _pass_a_kernel(adj_ref, x_ref, w1_ref, b1_ref, w2_ref,
                   g_ref, part_ref, ag_scr, g_scr):
    i = pl.program_id(0)
    hid = w1_ref.shape[1]

    @pl.when(i == 0)
    def _():
        ag_scr[:, :hid] = jnp.dot(x_ref[...], w1_ref[...],
                                  preferred_element_type=jnp.float32)
        ag_scr[:, hid:] = jnp.zeros_like(ag_scr[:, hid:])

    hp = jnp.dot(adj_ref[...], ag_scr[...],
                 preferred_element_type=jnp.float32)
    part_ref[...] = hp[:, hid:]
    h = jnp.maximum(hp[:, :hid] + b1_ref[...], 0.0)
    g_i = jnp.dot(h, w2_ref[...], preferred_element_type=jnp.float32)
    g_ref[...] = g_i
    g_scr[pl.ds(i * _BR, _BR), :] = g_i

    f_cur = (i * _BR // _BC) * _BC
    f_nxt = ((i + 1) * _BR // _BC) * _BC

    @pl.when(f_nxt > f_cur)
    def _():
        r0 = pl.multiple_of(f_cur, 8)
        ag_scr[pl.ds(r0, _BC), hid:] = g_scr[pl.ds(r0, _BC), :]


def _pass_b1_kernel(itb, ctb, ftb, adj_hbm, g_ref, part_ref, o_ref,
                    buf, sems):
    t = pl.program_id(0)
    nsteps = pl.num_programs(0)
    slot = jax.lax.rem(t, _NSLOT)

    def _copy(tt, s):
        r0 = pl.multiple_of(itb[tt] * _BR, 8)
        c0 = pl.multiple_of(ctb[tt], 128)
        return pltpu.make_async_copy(
            adj_hbm.at[pl.ds(r0, _BR), pl.ds(c0, _BC)],
            buf.at[s], sems.at[s])

    @pl.when(t == 0)
    def _():
        for k in range(_NSLOT - 1):
            @pl.when(k < nsteps)
            def _():
                _copy(k, k).start()

    @pl.when(t + _NSLOT - 1 < nsteps)
    def _():
        _copy(t + _NSLOT - 1, jax.lax.rem(t + _NSLOT - 1, _NSLOT)).start()

    _copy(t, slot).wait()

    contrib = jnp.dot(buf[slot],
                      g_ref[pl.ds(pl.multiple_of(ctb[t], 128), _BC), :],
                      preferred_element_type=jnp.float32)
    base = jnp.where(ftb[t] == 1, part_ref[...], o_ref[...])
    o_ref[...] = base + contrib


def _pass_b2_kernel(adj_hbm, g_ref, acc_ref, part_ref, b2_ref, o_ref,
                    buf, sems):
    t = pl.program_id(0)
    nsteps = pl.num_programs(0)
    slot = jax.lax.rem(t, _NSLOT)

    def _copy(tt, s):
        r0 = pl.multiple_of(tt * _BR, 8)
        return pltpu.make_async_copy(
            adj_hbm.at[pl.ds(r0, _BR), pl.ds(_CAP, _TW)],
            buf.at[s], sems.at[s])

    @pl.when(t == 0)
    def _():
        for k in range(_NSLOT - 1):
            @pl.when(k < nsteps)
            def _():
                _copy(k, k).start()

    @pl.when(t + _NSLOT - 1 < nsteps)
    def _():
        _copy(t + _NSLOT - 1, jax.lax.rem(t + _NSLOT - 1, _NSLOT)).start()

    _copy(t, slot).wait()

    contrib = jnp.dot(buf[slot], g_ref[pl.ds(_CAP, _TW), :],
                      preferred_element_type=jnp.float32)
    no_bulk = (t * _BR // _BC) * _BC == _CAP
    base = jnp.where(no_bulk, part_ref[...], acc_ref[...])
    z = jnp.maximum(base + contrib + b2_ref[...], 0.0)
    m = jnp.max(z, axis=1, keepdims=True)
    s = z - m
    lse = jnp.log(jnp.sum(jnp.exp(s), axis=1, keepdims=True))
    o_ref[...] = s - lse


def _bulk_tables(n):
    nb = n // _BR
    it, ct, ft = [], [], []
    for i in range(nb):
        f_i = (i * _BR // _BC) * _BC
        for k, c in enumerate(range(f_i, _CAP, _BC)):
            it.append(i)
            ct.append(c)
            ft.append(1 if k == 0 else 0)
    return (np.asarray(it, np.int32), np.asarray(ct, np.int32),
            np.asarray(ft, np.int32))


def kernel(x, adj, W1, b1, W2, b2):
    n, d_in = x.shape
    hid = W1.shape[1]
    classes = W2.shape[1]
    b1r = b1.reshape(1, hid)
    b2r = b2.reshape(1, classes)

    nb = n // _BR
    g, part = pl.pallas_call(
        _pass_a_kernel,
        grid=(nb,),
        in_specs=[
            pl.BlockSpec((_BR, n), lambda i: (i, 0)),
            pl.BlockSpec((n, d_in), lambda i: (0, 0)),
            pl.BlockSpec((d_in, hid), lambda i: (0, 0)),
            pl.BlockSpec((1, hid), lambda i: (0, 0)),
            pl.BlockSpec((hid, classes), lambda i: (0, 0)),
        ],
        out_specs=[
            pl.BlockSpec((_BR, classes), lambda i: (i, 0)),
            pl.BlockSpec((_BR, classes), lambda i: (i, 0)),
        ],
        out_shape=[
            jax.ShapeDtypeStruct((n, classes), jnp.float32),
            jax.ShapeDtypeStruct((n, classes), jnp.float32),
        ],
        scratch_shapes=[
            pltpu.VMEM((n, hid + classes), jnp.float32),
            pltpu.VMEM((n, classes), jnp.float32),
        ],
    )(adj, x, W1, b1r, W2)

    it, ct, ft = _bulk_tables(n)
    acc = pl.pallas_call(
        _pass_b1_kernel,
        grid_spec=pltpu.PrefetchScalarGridSpec(
            num_scalar_prefetch=3,
            grid=(it.shape[0],),
            in_specs=[
                pl.BlockSpec(memory_space=pl.ANY),
                pl.BlockSpec((n, classes), lambda t, it, ct, ft: (0, 0)),
                pl.BlockSpec((_BR, classes),
                             lambda t, it, ct, ft: (it[t], 0)),
            ],
            out_specs=pl.BlockSpec((_BR, classes),
                                   lambda t, it, ct, ft: (it[t], 0)),
            scratch_shapes=[
                pltpu.VMEM((_NSLOT, _BR, _BC), jnp.float32),
                pltpu.SemaphoreType.DMA((_NSLOT,)),
            ],
        ),
        out_shape=jax.ShapeDtypeStruct((n, classes), jnp.float32),
    )(jnp.asarray(it), jnp.asarray(ct), jnp.asarray(ft), adj, g, part)

    out = pl.pallas_call(
        _pass_b2_kernel,
        grid=(nb,),
        in_specs=[
            pl.BlockSpec(memory_space=pl.ANY),
            pl.BlockSpec((n, classes), lambda t: (0, 0)),
            pl.BlockSpec((_BR, classes), lambda t: (t, 0)),
            pl.BlockSpec((_BR, classes), lambda t: (t, 0)),
            pl.BlockSpec((1, classes), lambda t: (0, 0)),
        ],
        out_specs=pl.BlockSpec((_BR, classes), lambda t: (t, 0)),
        out_shape=jax.ShapeDtypeStruct((n, classes), jnp.float32),
        scratch_shapes=[
            pltpu.VMEM((_NSLOT, _BR, _TW), jnp.float32),
            pltpu.SemaphoreType.DMA((_NSLOT,)),
        ],
    )(adj, g, acc, part, b2r)
    return out


# speedup vs baseline: 1.6412x; 1.0512x over previous
import jax
import jax.numpy as jnp
import numpy as np
from jax.experimental import pallas as pl
from jax.experimental.pallas import tpu as pltpu

_BR = 400
_BC = 1280
_CAP = 8960
_TW = 1040
_NSLOT = 4


def _pass_a_kernel(adj_ref, x_ref, w1_ref, b1_ref, w2_ref,
                   g_ref, part_ref, ag_scr, g_scr):
    i = pl.program_id(0)
    hid = w1_ref.shape[1]

    @pl.when(i == 0)
    def _():
        ag_scr[:, :hid] = jnp.dot(x_ref[...], w1_ref[...],
                                  preferred_element_type=jnp.float32)
        ag_scr[:, hid:] = jnp.zeros_like(ag_scr[:, hid:])

    hp = jnp.dot(adj_ref[...], ag_scr[...],
                 preferred_element_type=jnp.float32)
    part_ref[...] = hp[:, hid:]
    h = jnp.maximum(hp[:, :hid] + b1_ref[...], 0.0)
    g_i = jnp.dot(h, w2_ref[...], preferred_element_type=jnp.float32)
    g_ref[...] = g_i
    g_scr[pl.ds(i * _BR, _BR), :] = g_i

    f_cur = (i * _BR // _BC) * _BC
    f_nxt = ((i + 1) * _BR // _BC) * _BC

    @pl.when(f_nxt > f_cur)
    def _():
        r0 = pl.multiple_of(f_cur, 8)
        ag_scr[pl.ds(r0, _BC), hid:] = g_scr[pl.ds(r0, _BC), :]


def _pass_b_kernel(itb, ctb, ftb, adj_hbm, g_ref, part_ref, b2_ref, o_ref,
                   buf, tbuf, sems):
    t = pl.program_id(0)
    nsteps = pl.num_programs(0)
    slot = jax.lax.rem(t, _NSLOT)

    def _copy(tt, s):
        r0 = pl.multiple_of(itb[tt] * _BR, 8)
        c0 = pl.multiple_of(ctb[tt], 128)
        is_tail = ctb[tt] == _CAP

        @pl.when(is_tail)
        def _():
            pltpu.make_async_copy(
                adj_hbm.at[pl.ds(r0, _BR), pl.ds(_CAP, _TW)],
                tbuf.at[s], sems.at[s]).start()

        @pl.when(jnp.logical_not(is_tail))
        def _():
            pltpu.make_async_copy(
                adj_hbm.at[pl.ds(r0, _BR), pl.ds(c0, _BC)],
                buf.at[s], sems.at[s]).start()

    def _wait(tt, s):
        r0 = pl.multiple_of(itb[tt] * _BR, 8)
        c0 = pl.multiple_of(ctb[tt], 128)
        is_tail = ctb[tt] == _CAP

        @pl.when(is_tail)
        def _():
            pltpu.make_async_copy(
                adj_hbm.at[pl.ds(r0, _BR), pl.ds(_CAP, _TW)],
                tbuf.at[s], sems.at[s]).wait()

        @pl.when(jnp.logical_not(is_tail))
        def _():
            pltpu.make_async_copy(
                adj_hbm.at[pl.ds(r0, _BR), pl.ds(c0, _BC)],
                buf.at[s], sems.at[s]).wait()

    @pl.when(t == 0)
    def _():
        for k in range(_NSLOT - 1):
            @pl.when(k < nsteps)
            def _():
                _copy(k, k)

    @pl.when(t + _NSLOT - 1 < nsteps)
    def _():
        _copy(t + _NSLOT - 1, jax.lax.rem(t + _NSLOT - 1, _NSLOT))

    _wait(t, slot)

    is_tail = ctb[t] == _CAP
    base = jnp.where(ftb[t] == 1, part_ref[...], o_ref[...])

    @pl.when(jnp.logical_not(is_tail))
    def _():
        contrib = jnp.dot(
            buf[slot],
            g_ref[pl.ds(pl.multiple_of(ctb[t], 128), _BC), :],
            preferred_element_type=jnp.float32)
        o_ref[...] = base + contrib

    @pl.when(is_tail)
    def _():
        contrib = jnp.dot(tbuf[slot], g_ref[pl.ds(_CAP, _TW), :],
                          preferred_element_type=jnp.float32)
        z = jnp.maximum(base + contrib + b2_ref[...], 0.0)
        m = jnp.max(z, axis=1, keepdims=True)
        s = z - m
        lse = jnp.log(jnp.sum(jnp.exp(s), axis=1, keepdims=True))
        o_ref[...] = s - lse


def _tile_tables(n):
    nb = n // _BR
    it, ct, ft = [], [], []
    for i in range(nb):
        f_i = (i * _BR // _BC) * _BC
        for k, c in enumerate(list(range(f_i, _CAP, _BC)) + [_CAP]):
            it.append(i)
            ct.append(c)
            ft.append(1 if k == 0 else 0)
    return (np.asarray(it, np.int32), np.asarray(ct, np.int32),
            np.asarray(ft, np.int32))


def kernel(x, adj, W1, b1, W2, b2):
    n, d_in = x.shape
    hid = W1.shape[1]
    classes = W2.shape[1]
    b1r = b1.reshape(1, hid)
    b2r = b2.reshape(1, classes)

    nb = n // _BR
    g, part = pl.pallas_call(
        _pass_a_kernel,
        grid=(nb,),
        in_specs=[
            pl.BlockSpec((_BR, n), lambda i: (i, 0)),
            pl.BlockSpec((n, d_in), lambda i: (0, 0)),
            pl.BlockSpec((d_in, hid), lambda i: (0, 0)),
            pl.BlockSpec((1, hid), lambda i: (0, 0)),
            pl.BlockSpec((hid, classes), lambda i: (0, 0)),
        ],
        out_specs=[
            pl.BlockSpec((_BR, classes), lambda i: (i, 0)),
            pl.BlockSpec((_BR, classes), lambda i: (i, 0)),
        ],
        out_shape=[
            jax.ShapeDtypeStruct((n, classes), jnp.float32),
            jax.ShapeDtypeStruct((n, classes), jnp.float32),
        ],
        scratch_shapes=[
            pltpu.VMEM((n, hid + classes), jnp.float32),
            pltpu.VMEM((n, classes), jnp.float32),
        ],
    )(adj, x, W1, b1r, W2)

    it, ct, ft = _tile_tables(n)
    out = pl.pallas_call(
        _pass_b_kernel,
        grid_spec=pltpu.PrefetchScalarGridSpec(
            num_scalar_prefetch=3,
            grid=(it.shape[0],),
            in_specs=[
                pl.BlockSpec(memory_space=pl.ANY),
                pl.BlockSpec((n, classes), lambda t, it, ct, ft: (0, 0)),
                pl.BlockSpec((_BR, classes),
                             lambda t, it, ct, ft: (it[t], 0)),
                pl.BlockSpec((1, classes), lambda t, it, ct, ft: (0, 0)),
            ],
            out_specs=pl.BlockSpec((_BR, classes),
                                   lambda t, it, ct, ft: (it[t], 0)),
            scratch_shapes=[
                pltpu.VMEM((_NSLOT, _BR, _BC), jnp.float32),
                pltpu.VMEM((_NSLOT, _BR, _TW), jnp.float32),
                pltpu.SemaphoreType.DMA((_NSLOT,)),
            ],
        ),
        out_shape=jax.ShapeDtypeStruct((n, classes), jnp.float32),
    )(jnp.asarray(it), jnp.asarray(ct), jnp.asarray(ft), adj, g, part, b2r)
    return out


# speedup vs baseline: 1.6641x; 1.0139x over previous
import jax
import jax.numpy as jnp
import numpy as np
from jax.experimental import pallas as pl
from jax.experimental.pallas import tpu as pltpu

_BR = 400
_BC = 1280
_CAP = 8960
_TW = 1040
_NSLOT = 4


def _pass_a_kernel(adj_ref, x_ref, w1_ref, b1_ref, w2_ref,
                   g_ref, part_ref, ag_scr, g_scr):
    i = pl.program_id(0)
    hid = w1_ref.shape[1]

    @pl.when(i == 0)
    def _():
        ag_scr[:, :hid] = jnp.dot(x_ref[...], w1_ref[...],
                                  preferred_element_type=jnp.float32)
        ag_scr[:, hid:] = jnp.zeros_like(ag_scr[:, hid:])

    hp = jnp.dot(adj_ref[...], ag_scr[...],
                 preferred_element_type=jnp.float32)
    part_ref[...] = hp[:, hid:]
    h = jnp.maximum(hp[:, :hid] + b1_ref[...], 0.0)
    g_i = jnp.dot(h, w2_ref[...], preferred_element_type=jnp.float32)
    g_ref[...] = g_i
    g_scr[pl.ds(i * _BR, _BR), :] = g_i

    f_cur = (i * _BR // _BC) * _BC
    f_nxt = ((i + 1) * _BR // _BC) * _BC

    @pl.when(f_nxt > f_cur)
    def _():
        r0 = pl.multiple_of(f_cur, 8)
        ag_scr[pl.ds(r0, _BC), hid:] = g_scr[pl.ds(r0, _BC), :]


def _pass_b_kernel(itb, ctb, ftb, wtb, adj_hbm, g_ref, part_ref, b2_ref,
                   o_ref, buf, tbuf, sems):
    t = pl.program_id(0)
    nsteps = pl.num_programs(0)
    slot = jax.lax.rem(t, _NSLOT)

    def _dma(tt, s):
        r0 = pl.multiple_of(itb[tt] * _BR, 8)
        c0 = pl.multiple_of(ctb[tt], 128)
        w = wtb[tt]
        wide = pltpu.make_async_copy(
            adj_hbm.at[pl.ds(r0, _BR), pl.ds(c0, 2 * _BC)],
            buf.at[s], sems.at[s])
        narrow = pltpu.make_async_copy(
            adj_hbm.at[pl.ds(r0, _BR), pl.ds(c0, _BC)],
            buf.at[s, :, pl.ds(0, _BC)], sems.at[s])
        tail = pltpu.make_async_copy(
            adj_hbm.at[pl.ds(r0, _BR), pl.ds(_CAP, _TW)],
            tbuf.at[s], sems.at[s])
        return w, wide, narrow, tail

    def _copy(tt, s):
        w, wide, narrow, tail = _dma(tt, s)
        pl.when(w == 0)(wide.start)
        pl.when(w == 1)(narrow.start)
        pl.when(w == 2)(tail.start)

    def _wait(tt, s):
        w, wide, narrow, tail = _dma(tt, s)
        pl.when(w == 0)(wide.wait)
        pl.when(w == 1)(narrow.wait)
        pl.when(w == 2)(tail.wait)

    @pl.when(t == 0)
    def _():
        for k in range(_NSLOT - 1):
            @pl.when(k < nsteps)
            def _():
                _copy(k, k)

    @pl.when(t + _NSLOT - 1 < nsteps)
    def _():
        _copy(t + _NSLOT - 1, jax.lax.rem(t + _NSLOT - 1, _NSLOT))

    _wait(t, slot)

    w = wtb[t]
    base = jnp.where(ftb[t] == 1, part_ref[...], o_ref[...])

    @pl.when(w == 0)
    def _():
        contrib = jnp.dot(
            buf[slot],
            g_ref[pl.ds(pl.multiple_of(ctb[t], 128), 2 * _BC), :],
            preferred_element_type=jnp.float32)
        o_ref[...] = base + contrib

    @pl.when(w == 1)
    def _():
        contrib = jnp.dot(
            buf[slot, :, :_BC],
            g_ref[pl.ds(pl.multiple_of(ctb[t], 128), _BC), :],
            preferred_element_type=jnp.float32)
        o_ref[...] = base + contrib

    @pl.when(w == 2)
    def _():
        contrib = jnp.dot(tbuf[slot], g_ref[pl.ds(_CAP, _TW), :],
                          preferred_element_type=jnp.float32)
        z = jnp.maximum(base + contrib + b2_ref[...], 0.0)
        m = jnp.max(z, axis=1, keepdims=True)
        s = z - m
        lse = jnp.log(jnp.sum(jnp.exp(s), axis=1, keepdims=True))
        o_ref[...] = s - lse


def _tile_tables(n):
    nb = n // _BR
    it, ct, ft, wt = [], [], [], []
    for i in range(nb):
        c = (i * _BR // _BC) * _BC
        tiles = []
        while _CAP - c >= 2 * _BC:
            tiles.append((c, 0))
            c += 2 * _BC
        if c < _CAP:
            tiles.append((c, 1))
        tiles.append((_CAP, 2))
        for k, (c, w) in enumerate(tiles):
            it.append(i)
            ct.append(c)
            ft.append(1 if k == 0 else 0)
            wt.append(w)
    return (np.asarray(it, np.int32), np.asarray(ct, np.int32),
            np.asarray(ft, np.int32), np.asarray(wt, np.int32))


def kernel(x, adj, W1, b1, W2, b2):
    n, d_in = x.shape
    hid = W1.shape[1]
    classes = W2.shape[1]
    b1r = b1.reshape(1, hid)
    b2r = b2.reshape(1, classes)

    nb = n // _BR
    g, part = pl.pallas_call(
        _pass_a_kernel,
        grid=(nb,),
        in_specs=[
            pl.BlockSpec((_BR, n), lambda i: (i, 0)),
            pl.BlockSpec((n, d_in), lambda i: (0, 0)),
            pl.BlockSpec((d_in, hid), lambda i: (0, 0)),
            pl.BlockSpec((1, hid), lambda i: (0, 0)),
            pl.BlockSpec((hid, classes), lambda i: (0, 0)),
        ],
        out_specs=[
            pl.BlockSpec((_BR, classes), lambda i: (i, 0)),
            pl.BlockSpec((_BR, classes), lambda i: (i, 0)),
        ],
        out_shape=[
            jax.ShapeDtypeStruct((n, classes), jnp.float32),
            jax.ShapeDtypeStruct((n, classes), jnp.float32),
        ],
        scratch_shapes=[
            pltpu.VMEM((n, hid + classes), jnp.float32),
            pltpu.VMEM((n, classes), jnp.float32),
        ],
    )(adj, x, W1, b1r, W2)

    it, ct, ft, wt = _tile_tables(n)
    out = pl.pallas_call(
        _pass_b_kernel,
        grid_spec=pltpu.PrefetchScalarGridSpec(
            num_scalar_prefetch=4,
            grid=(it.shape[0],),
            in_specs=[
                pl.BlockSpec(memory_space=pl.ANY),
                pl.BlockSpec((n, classes),
                             lambda t, it, ct, ft, wt: (0, 0)),
                pl.BlockSpec((_BR, classes),
                             lambda t, it, ct, ft, wt: (it[t], 0)),
                pl.BlockSpec((1, classes),
                             lambda t, it, ct, ft, wt: (0, 0)),
            ],
            out_specs=pl.BlockSpec((_BR, classes),
                                   lambda t, it, ct, ft, wt: (it[t], 0)),
            scratch_shapes=[
                pltpu.VMEM((_NSLOT, _BR, 2 * _BC), jnp.float32),
                pltpu.VMEM((_NSLOT, _BR, _TW), jnp.float32),
                pltpu.SemaphoreType.DMA((_NSLOT,)),
            ],
        ),
        out_shape=jax.ShapeDtypeStruct((n, classes), jnp.float32),
    )(jnp.asarray(it), jnp.asarray(ct), jnp.asarray(ft), jnp.asarray(wt),
      adj, g, part, b2r)
    return out


# speedup vs baseline: 1.6666x; 1.0015x over previous
import jax
import jax.numpy as jnp
import numpy as np
from jax.experimental import pallas as pl
from jax.experimental.pallas import tpu as pltpu

_BR = 400
_BC = 1280
_CAP = 8960
_TW = 1040
_NSLOT = 6


def _pass_a_kernel(adj_ref, x_ref, w1_ref, b1_ref, w2_ref,
                   g_ref, part_ref, ag_scr, g_scr):
    i = pl.program_id(0)
    hid = w1_ref.shape[1]

    @pl.when(i == 0)
    def _():
        ag_scr[:, :hid] = jnp.dot(x_ref[...], w1_ref[...],
                                  preferred_element_type=jnp.float32)
        ag_scr[:, hid:] = jnp.zeros_like(ag_scr[:, hid:])

    hp = jnp.dot(adj_ref[...], ag_scr[...],
                 preferred_element_type=jnp.float32)
    part_ref[...] = hp[:, hid:]
    h = jnp.maximum(hp[:, :hid] + b1_ref[...], 0.0)
    g_i = jnp.dot(h, w2_ref[...], preferred_element_type=jnp.float32)
    g_ref[...] = g_i
    g_scr[pl.ds(i * _BR, _BR), :] = g_i

    f_cur = (i * _BR // _BC) * _BC
    f_nxt = ((i + 1) * _BR // _BC) * _BC

    @pl.when(f_nxt > f_cur)
    def _():
        r0 = pl.multiple_of(f_cur, 8)
        ag_scr[pl.ds(r0, _BC), hid:] = g_scr[pl.ds(r0, _BC), :]


def _pass_b_kernel(itb, ctb, ftb, wtb, adj_hbm, g_ref, part_ref, b2_ref,
                   o_ref, buf, tbuf, sems):
    t = pl.program_id(0)
    nsteps = pl.num_programs(0)
    slot = jax.lax.rem(t, _NSLOT)

    def _dma(tt, s):
        r0 = pl.multiple_of(itb[tt] * _BR, 8)
        c0 = pl.multiple_of(ctb[tt], 128)
        w = wtb[tt]
        wide = pltpu.make_async_copy(
            adj_hbm.at[pl.ds(r0, _BR), pl.ds(c0, 2 * _BC)],
            buf.at[s], sems.at[s])
        narrow = pltpu.make_async_copy(
            adj_hbm.at[pl.ds(r0, _BR), pl.ds(c0, _BC)],
            buf.at[s, :, pl.ds(0, _BC)], sems.at[s])
        tail = pltpu.make_async_copy(
            adj_hbm.at[pl.ds(r0, _BR), pl.ds(_CAP, _TW)],
            tbuf.at[s], sems.at[s])
        return w, wide, narrow, tail

    def _copy(tt, s):
        w, wide, narrow, tail = _dma(tt, s)
        pl.when(w == 0)(wide.start)
        pl.when(w == 1)(narrow.start)
        pl.when(w == 2)(tail.start)

    def _wait(tt, s):
        w, wide, narrow, tail = _dma(tt, s)
        pl.when(w == 0)(wide.wait)
        pl.when(w == 1)(narrow.wait)
        pl.when(w == 2)(tail.wait)

    @pl.when(t == 0)
    def _():
        for k in range(_NSLOT - 1):
            @pl.when(k < nsteps)
            def _():
                _copy(k, k)

    @pl.when(t + _NSLOT - 1 < nsteps)
    def _():
        _copy(t + _NSLOT - 1, jax.lax.rem(t + _NSLOT - 1, _NSLOT))

    _wait(t, slot)

    w = wtb[t]
    base = jnp.where(ftb[t] == 1, part_ref[...], o_ref[...])

    @pl.when(w == 0)
    def _():
        contrib = jnp.dot(
            buf[slot],
            g_ref[pl.ds(pl.multiple_of(ctb[t], 128), 2 * _BC), :],
            preferred_element_type=jnp.float32)
        o_ref[...] = base + contrib

    @pl.when(w == 1)
    def _():
        contrib = jnp.dot(
            buf[slot, :, :_BC],
            g_ref[pl.ds(pl.multiple_of(ctb[t], 128), _BC), :],
            preferred_element_type=jnp.float32)
        o_ref[...] = base + contrib

    @pl.when(w == 2)
    def _():
        contrib = jnp.dot(tbuf[slot], g_ref[pl.ds(_CAP, _TW), :],
                          preferred_element_type=jnp.float32)
        z = jnp.maximum(base + contrib + b2_ref[...], 0.0)
        m = jnp.max(z, axis=1, keepdims=True)
        s = z - m
        lse = jnp.log(jnp.sum(jnp.exp(s), axis=1, keepdims=True))
        o_ref[...] = s - lse


def _tile_tables(n):
    nb = n // _BR
    it, ct, ft, wt = [], [], [], []
    for i in range(nb):
        c = (i * _BR // _BC) * _BC
        tiles = []
        while _CAP - c >= 2 * _BC:
            tiles.append((c, 0))
            c += 2 * _BC
        if c < _CAP:
            tiles.append((c, 1))
        tiles.append((_CAP, 2))
        for k, (c, w) in enumerate(tiles):
            it.append(i)
            ct.append(c)
            ft.append(1 if k == 0 else 0)
            wt.append(w)
    return (np.asarray(it, np.int32), np.asarray(ct, np.int32),
            np.asarray(ft, np.int32), np.asarray(wt, np.int32))


def kernel(x, adj, W1, b1, W2, b2):
    n, d_in = x.shape
    hid = W1.shape[1]
    classes = W2.shape[1]
    b1r = b1.reshape(1, hid)
    b2r = b2.reshape(1, classes)

    nb = n // _BR
    g, part = pl.pallas_call(
        _pass_a_kernel,
        grid=(nb,),
        in_specs=[
            pl.BlockSpec((_BR, n), lambda i: (i, 0)),
            pl.BlockSpec((n, d_in), lambda i: (0, 0)),
            pl.BlockSpec((d_in, hid), lambda i: (0, 0)),
            pl.BlockSpec((1, hid), lambda i: (0, 0)),
            pl.BlockSpec((hid, classes), lambda i: (0, 0)),
        ],
        out_specs=[
            pl.BlockSpec((_BR, classes), lambda i: (i, 0)),
            pl.BlockSpec((_BR, classes), lambda i: (i, 0)),
        ],
        out_shape=[
            jax.ShapeDtypeStruct((n, classes), jnp.float32),
            jax.ShapeDtypeStruct((n, classes), jnp.float32),
        ],
        scratch_shapes=[
            pltpu.VMEM((n, hid + classes), jnp.float32),
            pltpu.VMEM((n, classes), jnp.float32),
        ],
    )(adj, x, W1, b1r, W2)

    it, ct, ft, wt = _tile_tables(n)
    out = pl.pallas_call(
        _pass_b_kernel,
        grid_spec=pltpu.PrefetchScalarGridSpec(
            num_scalar_prefetch=4,
            grid=(it.shape[0],),
            in_specs=[
                pl.BlockSpec(memory_space=pl.ANY),
                pl.BlockSpec((n, classes),
                             lambda t, it, ct, ft, wt: (0, 0)),
                pl.BlockSpec((_BR, classes),
                             lambda t, it, ct, ft, wt: (it[t], 0)),
                pl.BlockSpec((1, classes),
                             lambda t, it, ct, ft, wt: (0, 0)),
            ],
            out_specs=pl.BlockSpec((_BR, classes),
                                   lambda t, it, ct, ft, wt: (it[t], 0)),
            scratch_shapes=[
                pltpu.VMEM((_NSLOT, _BR, 2 * _BC), jnp.float32),
                pltpu.VMEM((_NSLOT, _BR, _TW), jnp.float32),
                pltpu.SemaphoreType.DMA((_NSLOT,)),
            ],
        ),
        out_shape=jax.ShapeDtypeStruct((n, classes), jnp.float32),
    )(jnp.asarray(it), jnp.asarray(ct), jnp.asarray(ft), jnp.asarray(wt),
      adj, g, part, b2r)
    return out
